# baseline (speedup 1.0000x reference)
import functools

import jax
import jax.numpy as jnp
from jax import lax
from jax.experimental import pallas as pl
from jax.experimental.pallas import tpu as pltpu
from jax.experimental.pallas import tpu_sc as plsc

_B = 2
_N = 2048
_D = 128
_K = 16
_RT = 256
_NTILES = _B * _N // _RT

_NC, _NS, _L = 2, 16, 16
_NW = _NC * _NS
_RPW = _B * _N // _NW
_CH = 8
_NCH = _RPW // _CH

_LOG2_COEF = (
    -3.0283174810372375, 6.065830143177264, -5.2641104770701075,
    3.218832837050299, -1.2342631730323361, 0.26685882285942003,
    -0.024825606614202734,
)
_HALF_LN2 = 0.34657359027997264


def _cdist_body(x_ref, xt_ref, d2_ref):
    t = pl.program_id(0)
    xr = x_ref[0]
    xt = xt_ref[0]
    sq_all = jnp.sum(xt * xt, axis=0, keepdims=True)
    sq_r = jnp.sum(xr * xr, axis=1, keepdims=True)
    g = jax.lax.dot_general(
        xr, xt, (((1,), (0,)), ((), ())),
        preferred_element_type=jnp.float32,
        precision=jax.lax.Precision.HIGHEST)
    d2 = sq_r + sq_all - 2.0 * g
    rb = (t % (_N // _RT)) * _RT
    rows = rb + jax.lax.broadcasted_iota(jnp.int32, (_RT, _N), 0)
    cols = jax.lax.broadcasted_iota(jnp.int32, (_RT, _N), 1)
    d2_ref[...] = jnp.where(rows == cols, jnp.inf, jnp.maximum(d2, 1e-12))


def _cdist(X):
    xt = jnp.swapaxes(X, 1, 2)
    nrt = _N // _RT
    return pl.pallas_call(
        _cdist_body,
        grid=(_NTILES,),
        in_specs=[
            pl.BlockSpec((1, _RT, _D), lambda t: (t // nrt, t % nrt, 0)),
            pl.BlockSpec((1, _D, _N), lambda t: (t // nrt, 0, 0)),
        ],
        out_specs=pl.BlockSpec((_RT, _N), lambda t: (t, 0)),
        out_shape=jax.ShapeDtypeStruct((_B * _N, _N), jnp.float32),
    )(X, xt)


def _log2(q):
    bits = plsc.bitcast(q, jnp.int32)
    e = ((bits >> 23) & 0xFF) - 127
    mant = plsc.bitcast((bits & 0x7FFFFF) | 0x3F800000, jnp.float32)
    p = jnp.full((_L,), _LOG2_COEF[6], jnp.float32)
    for c in (_LOG2_COEF[5], _LOG2_COEF[4], _LOG2_COEF[3],
              _LOG2_COEF[2], _LOG2_COEF[1], _LOG2_COEF[0]):
        p = p * mant + c
    return e.astype(jnp.float32) + p


def _bcast_lane(x, lane_idx):
    idx = jnp.full((_L, 1), lane_idx, jnp.int32)
    return lax.gather(
        x, idx,
        dimension_numbers=lax.GatherDimensionNumbers(
            offset_dims=(), collapsed_slice_dims=(0,), start_index_map=(0,)),
        slice_sizes=(1,),
        mode=lax.GatherScatterMode.PROMISE_IN_BOUNDS)


def _merge16(T, v):
    cs, _ = plsc.sort_key_val(v, v, descending=True)
    m = jnp.minimum(T, cs)
    out, _ = plsc.sort_key_val(m, m)
    return out


def _sc_body(d2_hbm, out_hbm, buf, accv, sem):
    wid = lax.axis_index("s") * _NC + lax.axis_index("c")
    row0 = wid * _RPW
    lane = lax.broadcasted_iota(jnp.int32, (_L,), 0)
    inf_v = jnp.full((_L,), jnp.inf, jnp.float32)

    acc = jnp.zeros((_L,), jnp.float32)

    def chunk_body(ch, acc):
        pltpu.async_copy(
            d2_hbm.at[pl.ds(row0 + ch * _CH, _CH)], buf, sem).wait()
        Ts = [inf_v] * _CH

        def vstep(i, Ts):
            new = []
            for r in range(_CH):
                v = buf[r, pl.ds(i * _L, _L)]
                new.append(_merge16(Ts[r], v))
            return tuple(new)

        Ts = lax.fori_loop(0, _N // _L, vstep, tuple(Ts))

        for r in range(_CH):
            q = jnp.maximum(Ts[r], 1e-12)
            lg = _log2(q)
            l14 = _bcast_lane(lg, 14)
            acc = acc + jnp.where(lane <= 14, l14 - lg, 0.0)
        return acc

    acc = lax.fori_loop(0, _NCH, chunk_body, acc)
    accv[...] = acc * _HALF_LN2
    pltpu.sync_copy(accv, out_hbm.at[wid])


@functools.partial(
    pl.kernel,
    out_type=jax.ShapeDtypeStruct((_NW, _L), jnp.float32),
    mesh=plsc.VectorSubcoreMesh(core_axis_name="c", subcore_axis_name="s"),
    compiler_params=pltpu.CompilerParams(needs_layout_passes=False),
    scratch_types=[
        pltpu.VMEM((_CH, _N), jnp.float32),
        pltpu.VMEM((_L,), jnp.float32),
        pltpu.SemaphoreType.DMA,
    ],
)
def _sc_select(d2_hbm, out_hbm, buf, accv, sem):
    _sc_body(d2_hbm, out_hbm, buf, accv, sem)


def kernel(X):
    d2 = _cdist(X)
    parts = _sc_select(d2)
    s = jnp.sum(parts.reshape(_B, -1), axis=1)
    return (_K - 2) * _N / s


# speedup vs baseline: 1.2480x; 1.2480x over previous
import functools

import jax
import jax.numpy as jnp
from jax import lax
from jax.experimental import pallas as pl
from jax.experimental.pallas import tpu as pltpu
from jax.experimental.pallas import tpu_sc as plsc

_B = 2
_N = 2048
_D = 128
_K = 16
_RT = 256
_NT = _N // _RT

_NC, _NS, _L = 2, 16, 16
_NW = _NC * _NS
_RPW = _N // _NW
_CH = 8
_NCH = _RPW // _CH

_LOG2_COEF = (
    -3.0283174810372375, 6.065830143177264, -5.2641104770701075,
    3.218832837050299, -1.2342631730323361, 0.26685882285942003,
    -0.024825606614202734,
)
_HALF_LN2 = 0.34657359027997264


def _cdist_body(x_ref, xt_ref, d2_ref):
    t = pl.program_id(0)
    xr = x_ref[...]
    xt = xt_ref[...]
    sq_all = jnp.sum(xt * xt, axis=0, keepdims=True)
    sq_r = jnp.sum(xr * xr, axis=1, keepdims=True)
    g = jax.lax.dot_general(
        xr, xt, (((1,), (0,)), ((), ())),
        preferred_element_type=jnp.float32,
        precision=jax.lax.Precision.HIGHEST)
    d2 = sq_r + sq_all - 2.0 * g
    rows = t * _RT + jax.lax.broadcasted_iota(jnp.int32, (_RT, _N), 0)
    cols = jax.lax.broadcasted_iota(jnp.int32, (_RT, _N), 1)
    d2_ref[...] = jnp.where(rows == cols, jnp.inf, jnp.maximum(d2, 1e-12))


def _cdist(xb, xtb):
    return pl.pallas_call(
        _cdist_body,
        grid=(_NT,),
        in_specs=[
            pl.BlockSpec((_RT, _D), lambda t: (t, 0)),
            pl.BlockSpec((_D, _N), lambda t: (0, 0)),
        ],
        out_specs=pl.BlockSpec((_RT, _N), lambda t: (t, 0)),
        out_shape=jax.ShapeDtypeStruct((_N, _N), jnp.float32),
    )(xb, xtb)


def _log2(q):
    bits = plsc.bitcast(q, jnp.int32)
    e = ((bits >> 23) & 0xFF) - 127
    mant = plsc.bitcast((bits & 0x7FFFFF) | 0x3F800000, jnp.float32)
    p = jnp.full((_L,), _LOG2_COEF[6], jnp.float32)
    for c in (_LOG2_COEF[5], _LOG2_COEF[4], _LOG2_COEF[3],
              _LOG2_COEF[2], _LOG2_COEF[1], _LOG2_COEF[0]):
        p = p * mant + c
    return e.astype(jnp.float32) + p


def _bcast_lane(x, lane_idx):
    idx = jnp.full((_L, 1), lane_idx, jnp.int32)
    return lax.gather(
        x, idx,
        dimension_numbers=lax.GatherDimensionNumbers(
            offset_dims=(), collapsed_slice_dims=(0,), start_index_map=(0,)),
        slice_sizes=(1,),
        mode=lax.GatherScatterMode.PROMISE_IN_BOUNDS)


def _merge16(T, v):
    cs, _ = plsc.sort_key_val(v, v, descending=True)
    m = jnp.minimum(T, cs)
    out, _ = plsc.sort_key_val(m, m)
    return out


def _process_chunk(buf, acc, lane, inf_v):
    def vstep(i, Ts):
        return tuple(
            _merge16(Ts[r], buf[r, pl.ds(i * _L, _L)]) for r in range(_CH))

    Ts = lax.fori_loop(0, _N // _L, vstep, (inf_v,) * _CH)
    for r in range(_CH):
        lg = _log2(jnp.maximum(Ts[r], 1e-12))
        l14 = _bcast_lane(lg, 14)
        acc = acc + jnp.where(lane <= 14, l14 - lg, 0.0)
    return acc


def _sc_body(d2_hbm, out_hbm, bufa, bufb, accv, sema, semb):
    wid = lax.axis_index("s") * _NC + lax.axis_index("c")
    row0 = wid * _RPW
    lane = lax.broadcasted_iota(jnp.int32, (_L,), 0)
    inf_v = jnp.full((_L,), jnp.inf, jnp.float32)

    bufs = (bufa, bufb)
    sems = (sema, semb)

    def start(ch):
        return pltpu.async_copy(
            d2_hbm.at[pl.ds(row0 + ch * _CH, _CH)], bufs[ch % 2],
            sems[ch % 2])

    acc = jnp.zeros((_L,), jnp.float32)
    pending = start(0)
    for ch in range(_NCH):
        cur = pending
        pending = start(ch + 1) if ch + 1 < _NCH else None
        cur.wait()
        acc = _process_chunk(bufs[ch % 2], acc, lane, inf_v)

    accv[...] = acc * _HALF_LN2
    pltpu.sync_copy(accv, out_hbm.at[wid])


@functools.partial(
    pl.kernel,
    out_type=jax.ShapeDtypeStruct((_NW, _L), jnp.float32),
    mesh=plsc.VectorSubcoreMesh(core_axis_name="c", subcore_axis_name="s"),
    compiler_params=pltpu.CompilerParams(needs_layout_passes=False),
    scratch_types=[
        pltpu.VMEM((_CH, _N), jnp.float32),
        pltpu.VMEM((_CH, _N), jnp.float32),
        pltpu.VMEM((_L,), jnp.float32),
        pltpu.SemaphoreType.DMA,
        pltpu.SemaphoreType.DMA,
    ],
)
def _sc_select(d2_hbm, out_hbm, bufa, bufb, accv, sema, semb):
    _sc_body(d2_hbm, out_hbm, bufa, bufb, accv, sema, semb)


def kernel(X):
    xt = jnp.swapaxes(X, 1, 2)
    s = []
    for b in range(_B):
        d2 = _cdist(X[b], xt[b])
        parts = _sc_select(d2)
        s.append(jnp.sum(parts))
    return (_K - 2) * _N / jnp.stack(s)
